# baseline (speedup 1.0000x reference)
import functools

import jax
import jax.numpy as jnp
from jax import lax
from jax.experimental import pallas as pl
from jax.experimental.pallas import tpu as pltpu
from jax.experimental.pallas import tpu_sc as plsc

_MAX_STAGES = 100000
_FEAT = 64
_BATCH = 16384
_HIST = 50
_B_TOTAL = _BATCH * _HIST

_FUSE_ROWS = 2000


def _fuse_body(scale_ref, base_ref, resid_ref, out_ref):
    out_ref[...] = base_ref[...] * scale_ref[0, 0] + resid_ref[...]


def _fuse_tables(scale, base_pe, residual_w):
    grid = _MAX_STAGES // _FUSE_ROWS
    return pl.pallas_call(
        _fuse_body,
        grid=(grid,),
        in_specs=[
            pl.BlockSpec(memory_space=pltpu.SMEM),
            pl.BlockSpec((_FUSE_ROWS, _FEAT), lambda i: (i, 0)),
            pl.BlockSpec((_FUSE_ROWS, _FEAT), lambda i: (i, 0)),
        ],
        out_specs=pl.BlockSpec((_FUSE_ROWS, _FEAT), lambda i: (i, 0)),
        out_shape=jax.ShapeDtypeStruct((_MAX_STAGES, _FEAT), jnp.float32),
    )(scale.reshape(1, 1), base_pe, residual_w)


_info = plsc.get_sparse_core_info()
_NC, _NS = _info.num_cores, _info.num_subcores
_NW = _NC * _NS
_NTASK = _HIST * 128
_TPW = _NTASK // _NW
_NPAIR = _TPW // 2

_sc_mesh = plsc.VectorSubcoreMesh(core_axis_name="c", subcore_axis_name="s")


@functools.partial(
    pl.kernel,
    mesh=_sc_mesh,
    out_type=jax.ShapeDtypeStruct((51200, 1024), jnp.float32),
    scratch_types=[
        pltpu.VMEM((128,), jnp.int32),
        pltpu.VMEM((128,), jnp.int32),
        pltpu.VMEM((128, _FEAT), jnp.float32),
        pltpu.VMEM((128, _FEAT), jnp.float32),
        pltpu.VMEM((8192,), jnp.float32),
        pltpu.VMEM((8192,), jnp.float32),
        pltpu.SemaphoreType.DMA,
        pltpu.SemaphoreType.DMA,
        pltpu.SemaphoreType.DMA,
        pltpu.SemaphoreType.DMA,
    ],
    compiler_params=pltpu.CompilerParams(
        use_tc_tiling_on_sc=False, needs_layout_passes=False),
)
def _sc_gather(idx_hbm, tab_hbm, out_hbm, idx0, idx1, rows0, rows1,
               tile0, tile1, gsem0, gsem1, wsem0, wsem1):
    wid = lax.axis_index("s") * _NC + lax.axis_index("c")
    base = wid * _TPW

    lane = lax.iota(jnp.int32, 16)

    def g_start(t, idxbuf, rowsbuf, sem):
        pltpu.sync_copy(idx_hbm.at[t], idxbuf)
        pltpu.async_copy(tab_hbm.at[idxbuf], rowsbuf, sem)

    def g_wait(idxbuf, rowsbuf, sem):
        pltpu.make_async_copy(tab_hbm.at[idxbuf], rowsbuf, sem).wait()

    def w_each(t, tilebuf, sem, fn):
        h = t // 128
        c = t - h * 128
        for r in range(8):
            fn(tilebuf.at[pl.ds(r * 1024, 1024)], out_hbm.at[(h * 8 + r) * 128 + c], sem)

    def w_start(t, tilebuf, sem):
        w_each(t, tilebuf, sem, lambda s, d, m: pltpu.async_copy(s, d, m))

    def w_wait(t, tilebuf, sem):
        w_each(t, tilebuf, sem, lambda s, d, m: pltpu.make_async_copy(s, d, m).wait())

    def transpose(rowsbuf, tilebuf):
        def fbody(f8, carry):
            for df in range(8):
                f = f8 * 8 + df
                fv = jnp.full((16,), 0, jnp.int32) + f
                for g in range(8):
                    v = plsc.load_gather(rowsbuf, [lane + g * 16, fv])
                    tilebuf[pl.ds(f * 128 + g * 16, 16)] = v
            return carry

        lax.fori_loop(0, 8, fbody, 0)

    def task(t, idxbuf, rowsbuf, tilebuf, gsem, wsem, drain):
        g_wait(idxbuf, rowsbuf, gsem)
        if drain:
            w_wait(t, tilebuf, wsem)
        transpose(rowsbuf, tilebuf)
        w_start(t, tilebuf, wsem)

    g_start(base + 0, idx0, rows0, gsem0)
    g_start(base + 1, idx1, rows1, gsem1)
    task(base + 0, idx0, rows0, tile0, gsem0, wsem0, drain=False)
    g_start(base + 2, idx0, rows0, gsem0)
    task(base + 1, idx1, rows1, tile1, gsem1, wsem1, drain=False)
    g_start(base + 3, idx1, rows1, gsem1)

    def body(p, carry):
        a = base + 2 * p
        task(a, idx0, rows0, tile0, gsem0, wsem0, drain=True)
        g_start(a + 2, idx0, rows0, gsem0)
        task(a + 1, idx1, rows1, tile1, gsem1, wsem1, drain=True)
        g_start(a + 3, idx1, rows1, gsem1)
        return carry

    lax.fori_loop(1, _NPAIR - 1, body, 0)

    a = base + _TPW - 2
    task(a, idx0, rows0, tile0, gsem0, wsem0, drain=True)
    task(a + 1, idx1, rows1, tile1, gsem1, wsem1, drain=True)
    w_wait(a, tile0, wsem0)
    w_wait(a + 1, tile1, wsem1)


def kernel(stage_labels, base_pe, residual_w, scale):
    combined = _fuse_tables(scale, base_pe, residual_w)
    idx = jnp.swapaxes(stage_labels, 0, 1).reshape(_NTASK, 128).astype(jnp.int32)
    out = _sc_gather(idx, combined)
    out5 = out.reshape(_HIST, 8, 128, 8, 128)
    return out5.transpose(2, 4, 0, 1, 3).reshape(_BATCH, _HIST, _FEAT)


# speedup vs baseline: 5.8163x; 5.8163x over previous
import functools

import jax
import jax.numpy as jnp
from jax import lax
from jax.experimental import pallas as pl
from jax.experimental.pallas import tpu as pltpu
from jax.experimental.pallas import tpu_sc as plsc

_MAX_STAGES = 100000
_FEAT = 64
_BATCH = 16384
_HIST = 50
_B_TOTAL = _BATCH * _HIST

_FUSE_COLS = 8192


def _fuse_body(scale_ref, baseT_ref, residT_ref, out_ref):
    fused = baseT_ref[...] * scale_ref[0, 0] + residT_ref[...]
    out_ref[...] = fused.T


def _fuse_tables(scale, base_pe, residual_w):
    grid = -(-_MAX_STAGES // _FUSE_COLS)
    out128 = pl.pallas_call(
        _fuse_body,
        grid=(grid,),
        in_specs=[
            pl.BlockSpec(memory_space=pltpu.SMEM),
            pl.BlockSpec((_FEAT, _FUSE_COLS), lambda i: (0, i)),
            pl.BlockSpec((_FEAT, _FUSE_COLS), lambda i: (0, i)),
        ],
        out_specs=pl.BlockSpec((_FUSE_COLS, _FEAT), lambda i: (i, 0)),
        out_shape=jax.ShapeDtypeStruct((_MAX_STAGES, _FEAT), jnp.float32),
    )(scale.reshape(1, 1), base_pe.T, residual_w.T)
    return out128


_info = plsc.get_sparse_core_info()
_NC, _NS = _info.num_cores, _info.num_subcores
_NW = _NC * _NS
_NTASK = _HIST * 64
_TPW = _NTASK // _NW
_NQUAD = _TPW // 4

_sc_mesh = plsc.VectorSubcoreMesh(core_axis_name="c", subcore_axis_name="s")


@functools.partial(
    pl.kernel,
    mesh=_sc_mesh,
    out_type=jax.ShapeDtypeStruct((52428800,), jnp.float32),
    scratch_types=[
        pltpu.VMEM((4, 256), jnp.int32),
        pltpu.VMEM((256, _FEAT), jnp.float32),
        pltpu.VMEM((256, _FEAT), jnp.float32),
        pltpu.VMEM((256, _FEAT), jnp.float32),
        pltpu.VMEM((256, _FEAT), jnp.float32),
        pltpu.VMEM((16384,), jnp.float32),
        pltpu.VMEM((16384,), jnp.float32),
        pltpu.SemaphoreType.DMA,
        pltpu.SemaphoreType.DMA,
        pltpu.SemaphoreType.DMA,
        pltpu.SemaphoreType.DMA,
        pltpu.SemaphoreType.DMA,
        pltpu.SemaphoreType.DMA,
    ],
    compiler_params=pltpu.CompilerParams(
        use_tc_tiling_on_sc=False, needs_layout_passes=False),
)
def _sc_gather(idx_hbm, tab_hbm, out_hbm, idxv, rows0, rows1, rows2, rows3,
               tile0, tile1, gsem0, gsem1, gsem2, gsem3, wsem0, wsem1):
    wid = lax.axis_index("s") * _NC + lax.axis_index("c")
    base = wid * _TPW

    lane = lax.iota(jnp.int32, 16)
    rows = [rows0, rows1, rows2, rows3]
    gsems = [gsem0, gsem1, gsem2, gsem3]
    tiles = [tile0, tile1]
    wsems = [wsem0, wsem1]

    def g_start(t, k):
        pltpu.sync_copy(idx_hbm.at[t], idxv.at[k])
        pltpu.async_copy(tab_hbm.at[idxv.at[k]], rows[k], gsems[k])

    def g_wait(k):
        pltpu.make_async_copy(tab_hbm.at[idxv.at[k]], rows[k], gsems[k]).wait()

    def w_each(t, j, fn):
        h = t // 64
        c = (t - h * 64) * 2
        for r in range(8):
            off = pl.multiple_of(((h * 8 + r) * 128 + c) * 1024, 2048)
            fn(tiles[j].at[pl.ds(r * 2048, 2048)], out_hbm.at[pl.ds(off, 2048)],
               wsems[j])

    def w_start(t, j):
        w_each(t, j, lambda sr, d, m: pltpu.async_copy(sr, d, m))

    def w_wait(t, j):
        w_each(t, j, lambda sr, d, m: pltpu.make_async_copy(sr, d, m).wait())

    jv = [lane + g * 16 for g in range(16)]
    dv = [lane + (g >> 3) * 1024 + (g & 7) * 16 for g in range(16)]

    def transpose(k, j):
        rowsbuf, tilebuf = rows[k], tiles[j]

        @plsc.parallel_loop(0, _FEAT, step=1, unroll=8)
        def fbody(i):
            tv = (lane + i) & 15
            fv = tv + (i & 48)
            dstf = (tv & 7) * 128 + (tv >> 3) * 2048 + (i & 48) * 256
            for g in range(16):
                v = plsc.load_gather(rowsbuf, [jv[g], fv])
                plsc.store_scatter(tilebuf, [dstf + dv[g]], v)

    def task(t, k, j, drain):
        g_wait(k)
        if drain:
            w_wait(t, j)
        transpose(k, j)
        w_start(t, j)

    for k in range(4):
        g_start(base + k, k)
    task(base + 0, 0, 0, drain=False)
    g_start(base + 4, 0)
    task(base + 1, 1, 1, drain=False)
    g_start(base + 5, 1)
    task(base + 2, 2, 0, drain=True)
    g_start(base + 6, 2)
    task(base + 3, 3, 1, drain=True)
    g_start(base + 7, 3)

    def body(p, carry):
        a = base + 4 * p
        for k in range(4):
            task(a + k, k, k & 1, drain=True)
            g_start(a + k + 4, k)
        return carry

    lax.fori_loop(1, _NQUAD - 1, body, 0)

    a = base + _TPW - 4
    for k in range(4):
        task(a + k, k, k & 1, drain=True)
    w_wait(a + 2, 0)
    w_wait(a + 3, 1)


def kernel(stage_labels, base_pe, residual_w, scale):
    combined = _fuse_tables(scale, base_pe, residual_w)
    idx = jnp.swapaxes(stage_labels, 0, 1).reshape(_NTASK, 256).astype(jnp.int32)
    out = _sc_gather(idx, combined)
    out5 = out.reshape(_HIST, 8, 128, 8, 128)
    return out5.transpose(2, 4, 0, 1, 3).reshape(_BATCH, _HIST, _FEAT)
